# baseline (speedup 1.0000x reference)
import functools

import jax
import jax.numpy as jnp
from jax import lax
from jax.experimental import pallas as pl
from jax.experimental.pallas import tpu as pltpu
from jax.experimental.pallas import tpu_sc as plsc

BATCH = 4096
SEQ = 200
EMBED_DIM = 64
VOCAB = 1000000
NW = 32
BBLK = BATCH // NW
SBLK = 8
N_SBLK = SEQ // SBLK


def _make_emb_kernel():
    mesh = plsc.VectorSubcoreMesh(core_axis_name="c", subcore_axis_name="s")

    @functools.partial(
        pl.kernel,
        mesh=mesh,
        out_type=jax.ShapeDtypeStruct((SEQ, EMBED_DIM, BATCH), jnp.float32),
        scratch_types=[
            pltpu.VMEM((SBLK, BBLK), jnp.int32),
            pltpu.VMEM((2, BBLK), jnp.int32),
            pltpu.VMEM((2, BBLK, 128), jnp.float32),
            pltpu.VMEM((2, EMBED_DIM, BBLK), jnp.float32),
            pltpu.SemaphoreType.DMA,
            pltpu.SemaphoreType.DMA,
            pltpu.SemaphoreType.DMA,
            pltpu.SemaphoreType.DMA,
        ],
        compiler_params=pltpu.CompilerParams(use_tc_tiling_on_sc=True,
                                             needs_layout_passes=False),
    )
    def emb(sent_t, tview, out_hbm, idxblk, pairv, gbuf, obuf,
            gsem0, gsem1, wsem0, wsem1):
        gsems = (gsem0, gsem1)
        wsems = (wsem0, wsem1)
        wid = lax.axis_index("s") * 2 + lax.axis_index("c")
        bbase = wid * BBLK

        iota16 = lax.iota(jnp.int32, 16)

        def prep_pair(j, buf):
            for k in range(BBLK // 16):
                v = idxblk[j, pl.ds(k * 16, 16)]
                pairv[buf, pl.ds(k * 16, 16)] = lax.shift_right_logical(v, 1)

        def gather_start(buf):
            pltpu.async_copy(tview.at[pairv.at[buf]], gbuf.at[buf],
                             gsems[buf])

        def gather_wait(buf):
            pltpu.make_async_copy(tview.at[pairv.at[buf]], gbuf.at[buf],
                                  gsems[buf]).wait()

        def transpose(j, buf):
            rows = [iota16 + (k * 16) for k in range(BBLK // 16)]
            colbase = []
            for k in range(BBLK // 16):
                v = idxblk[j, pl.ds(k * 16, 16)]
                colbase.append(lax.shift_left(
                    lax.bitwise_and(v, jnp.int32(1)), 6))

            def body(d, cols):
                for k in range(BBLK // 16):
                    vals = plsc.load_gather(gbuf.at[buf], [rows[k], cols[k]])
                    obuf[buf, d, pl.ds(k * 16, 16)] = vals
                return tuple(c + 1 for c in cols)

            lax.fori_loop(0, EMBED_DIM, body, tuple(colbase))

        def write_start(s, buf):
            pltpu.async_copy(
                obuf.at[buf],
                out_hbm.at[s, :, pl.ds(bbase, BBLK)], wsems[buf])

        def write_wait(s, buf):
            pltpu.make_async_copy(
                obuf.at[buf],
                out_hbm.at[s, :, pl.ds(bbase, BBLK)], wsems[buf]).wait()

        def sblock(blk, carry):
            pltpu.sync_copy(
                sent_t.at[pl.ds(blk * SBLK, SBLK), pl.ds(bbase, BBLK)],
                idxblk)
            s0 = blk * SBLK
            prep_pair(0, 0)
            gather_start(0)
            for j in range(SBLK):
                buf = j % 2
                nbuf = 1 - buf
                if j + 1 < SBLK:
                    prep_pair(j + 1, nbuf)
                    gather_start(nbuf)
                gather_wait(buf)
                if j >= 2:
                    write_wait(s0 + j - 2, buf)
                transpose(j, buf)
                write_start(s0 + j, buf)
            write_wait(s0 + SBLK - 2, 0)
            write_wait(s0 + SBLK - 1, 1)
            return carry

        lax.fori_loop(0, N_SBLK, sblock, 0)

    return emb


_emb = _make_emb_kernel()


def kernel(sentence, table):
    tview = jnp.reshape(table, (VOCAB // 2, 128))
    sent_t = jnp.transpose(sentence)
    y = _emb(sent_t, tview)
    return jnp.transpose(y, (2, 0, 1))
